# baseline (speedup 1.0000x reference)
import functools

import jax
import jax.numpy as jnp
from jax import lax
from jax.experimental import pallas as pl
from jax.experimental.pallas import tpu as pltpu
from jax.experimental.pallas import tpu_sc as plsc

S, D, DFF, E, K = 2048, 1024, 4096, 8, 2
TR = 256
NT = (K * S) // TR + E
P = NT * TR
MAX_TILES_PER_E = S // TR
DFB = 1024
NJ = DFF // DFB

NC, NS, NL = 2, 16, 16
NW = NC * NS


def _gate_body(x_ref, wg_ref, bg_ref,
               p0_ref, p1_ref, pp0_ref, pp1_ref, dlt_ref, nt_ref, gs_ref):
    logits = lax.dot_general(
        wg_ref[...], x_ref[...], (((1,), (1,)), ((), ())),
        preferred_element_type=jnp.float32)
    logits = logits + bg_ref[:, 0:1]
    m0 = jnp.max(logits, axis=0)
    i0 = jnp.argmax(logits, axis=0)
    rows = lax.broadcasted_iota(jnp.int32, logits.shape, 0)
    masked = jnp.where(rows == i0[None, :], -1e30, logits)
    m1 = jnp.max(masked, axis=0)
    i1 = jnp.argmax(masked, axis=0)
    p0 = 1.0 / (1.0 + jnp.exp(m1 - m0))
    p0_ref[...] = p0
    p1_ref[...] = 1.0 - p0

    e_all = jnp.concatenate([i0, i1])
    oh = (e_all[None, :] == lax.broadcasted_iota(jnp.int32, (E, 2 * S), 0)
          ).astype(jnp.int32)
    c = oh
    sh = 1
    while sh < 2 * S:
        z = jnp.zeros((E, sh), jnp.int32)
        c = c + jnp.concatenate([z, c[:, : 2 * S - sh]], axis=1)
        sh *= 2
    rank = jnp.sum(oh * c, axis=0) - 1
    counts = c[:, -1:]
    ntiles = (counts + (TR - 1)) // TR
    tri = (lax.broadcasted_iota(jnp.int32, (E, E), 1)
           < lax.broadcasted_iota(jnp.int32, (E, E), 0)).astype(jnp.float32)
    ex = lax.dot_general(tri, ntiles.astype(jnp.float32),
                         (((1,), (0,)), ((), ())),
                         preferred_element_type=jnp.float32)
    g_start = (ex * TR).astype(jnp.int32)
    gsel = jnp.sum(oh * g_start, axis=0)
    ppos = gsel + rank
    tok = jnp.concatenate([jnp.arange(S, dtype=jnp.int32)] * 2)
    dlt_ref[...] = tok - (ppos & (S - 1))
    pp0_ref[...] = ppos[:S]
    pp1_ref[...] = ppos[S:]
    nt_ref[...] = jnp.broadcast_to(ntiles, (E, 128))
    gs_ref[...] = jnp.broadcast_to(g_start, (E, 128))


def _gate(x2, Wg, bg):
    bgb = jnp.broadcast_to(bg[:, None], (E, 128)).astype(jnp.float32)
    return pl.pallas_call(
        _gate_body,
        out_shape=(
            jax.ShapeDtypeStruct((S,), jnp.float32),
            jax.ShapeDtypeStruct((S,), jnp.float32),
            jax.ShapeDtypeStruct((S,), jnp.int32),
            jax.ShapeDtypeStruct((S,), jnp.int32),
            jax.ShapeDtypeStruct((2 * S,), jnp.int32),
            jax.ShapeDtypeStruct((E, 128), jnp.int32),
            jax.ShapeDtypeStruct((E, 128), jnp.int32),
        ),
    )(x2, Wg, bgb)


RPW = P // NW
GCH = 32
GNCH = RPW // GCH


def _sc_gather_body(x_hbm, idx_hbm, xs_hbm, idx_v, b0, b1,
                    g0, g1, st0, st1):
    wid = lax.axis_index("s") * NC + lax.axis_index("c")
    base = wid * RPW
    pltpu.sync_copy(idx_hbm.at[pl.ds(base, RPW)], idx_v)
    bufs, gsem, ssem = (b0, b1), (g0, g1), (st0, st1)

    def gather(c):
        return pltpu.async_copy(
            x_hbm.at[idx_v.at[pl.ds(c * GCH, GCH)]], bufs[c % 2], gsem[c % 2])

    cps = {0: gather(0), 1: gather(1)}
    sts = {}
    for c in range(GNCH):
        cps[c].wait()
        sts[c] = pltpu.async_copy(
            bufs[c % 2], xs_hbm.at[pl.ds(base + c * GCH, GCH)], ssem[c % 2])
        if 1 <= c < GNCH - 1:
            sts[c - 1].wait()
            cps[c + 1] = gather(c + 1)
    if GNCH >= 2:
        sts[GNCH - 2].wait()
    sts[GNCH - 1].wait()


@functools.cache
def _sc_gather_fn():
    return functools.partial(
        pl.kernel,
        mesh=plsc.VectorSubcoreMesh(core_axis_name="c", subcore_axis_name="s"),
        compiler_params=pltpu.CompilerParams(use_tc_tiling_on_sc=True),
        out_type=jax.ShapeDtypeStruct((P, D), jnp.float32),
        scratch_types=[
            pltpu.VMEM((RPW,), jnp.int32),
            pltpu.VMEM((GCH, D), jnp.float32),
            pltpu.VMEM((GCH, D), jnp.float32),
            pltpu.SemaphoreType.DMA,
            pltpu.SemaphoreType.DMA,
            pltpu.SemaphoreType.DMA,
            pltpu.SemaphoreType.DMA,
        ],
    )(_sc_gather_body)


def _ffn_body(ntiles_ref, gstart_ref, xs_ref, w1_ref, b1_ref, w2_ref, b2_ref,
              ys_ref, acc_ref, sem):
    e = pl.program_id(0)
    j = pl.program_id(1)
    nt = ntiles_ref[e]
    gs = pl.multiple_of(gstart_ref[e], TR)
    for t in range(MAX_TILES_PER_E):
        @pl.when(t < nt)
        def _():
            xt = xs_ref[pl.ds(gs + t * TR, TR), :]
            h = lax.dot_general(xt, w1_ref[...], (((1,), (1,)), ((), ())),
                                preferred_element_type=jnp.float32)
            h = h + b1_ref[0]
            h = 0.5 * h * (1.0 + lax.erf(h * 0.7071067811865476))
            y = lax.dot_general(h, w2_ref[...], (((1,), (1,)), ((), ())),
                                preferred_element_type=jnp.float32)

            @pl.when(j == 0)
            def _():
                acc_ref[t] = y + b2_ref[0]

            @pl.when(j > 0)
            def _():
                acc_ref[t] = acc_ref[t] + y

    @pl.when(j == NJ - 1)
    def _():
        for t in range(MAX_TILES_PER_E):
            @pl.when(t < nt)
            def _():
                pltpu.make_async_copy(
                    acc_ref.at[t], ys_ref.at[pl.ds(gs + t * TR, TR), :], sem
                ).start()
        for t in range(MAX_TILES_PER_E):
            @pl.when(t < nt)
            def _():
                pltpu.make_async_copy(
                    acc_ref.at[t], ys_ref.at[pl.ds(gs + t * TR, TR), :], sem
                ).wait()


def _ffn(ntiles, gstart, xs, W1, b1, W2, b2):
    b1r = b1.reshape(E, 1, DFF)
    b2r = b2.reshape(E, 1, D)
    grid_spec = pltpu.PrefetchScalarGridSpec(
        num_scalar_prefetch=2,
        grid=(E, NJ),
        in_specs=[
            pl.BlockSpec((P, D), lambda e, j, nt, gsr: (0, 0)),
            pl.BlockSpec((1, DFB, D), lambda e, j, nt, gsr: (e, j, 0)),
            pl.BlockSpec((1, 1, DFB), lambda e, j, nt, gsr: (e, 0, j)),
            pl.BlockSpec((1, D, DFB), lambda e, j, nt, gsr: (e, 0, j)),
            pl.BlockSpec((1, 1, D), lambda e, j, nt, gsr: (e, 0, 0)),
        ],
        out_specs=pl.BlockSpec(memory_space=pl.ANY),
        scratch_shapes=[
            pltpu.VMEM((MAX_TILES_PER_E, TR, D), jnp.float32),
            pltpu.SemaphoreType.DMA,
        ],
    )

    def body(ntiles_ref, gstart_ref, xs_ref, w1_ref, b1_ref, w2_ref, b2_ref,
             ys_ref, acc_ref, sem):
        _ffn_body(ntiles_ref, gstart_ref, xs_ref,
                  w1_ref.at[0], b1_ref.at[0], w2_ref.at[0], b2_ref.at[0],
                  ys_ref, acc_ref, sem)

    return pl.pallas_call(
        body,
        grid_spec=grid_spec,
        out_shape=jax.ShapeDtypeStruct((P, D), jnp.float32),
        compiler_params=pltpu.CompilerParams(
            dimension_semantics=("arbitrary", "arbitrary")),
    )(ntiles, gstart, xs, W1, b1r, W2, b2r)


TPW = S // NW
CCH = 16
CNCH = TPW // CCH
NVEC = D // NL


def _sc_combine_body(ys_hbm, pint_hbm, p0_hbm, p1_hbm, out_hbm,
                     ii_v, p0_v, p1_v, G0, G1, O0, O1, g0, g1, st0, st1):
    wid = lax.axis_index("s") * NC + lax.axis_index("c")
    base = wid * TPW
    pltpu.sync_copy(pint_hbm.at[pl.ds(2 * base, 2 * TPW)], ii_v)
    pltpu.sync_copy(p0_hbm.at[pl.ds(base, TPW), :], p0_v)
    pltpu.sync_copy(p1_hbm.at[pl.ds(base, TPW), :], p1_v)
    G, O, gsem, ssem = (G0, G1), (O0, O1), (g0, g1), (st0, st1)

    def gather(c):
        return pltpu.async_copy(
            ys_hbm.at[ii_v.at[pl.ds(c * 2 * CCH, 2 * CCH)]], G[c % 2],
            gsem[c % 2])

    cps = {0: gather(0), 1: gather(1)}
    sts = {}
    for c in range(CNCH):
        cps[c].wait()
        if c - 2 >= 0:
            sts[c - 2].wait()
        Gc, Oc = G[c % 2], O[c % 2]
        for t in range(CCH):
            w0 = p0_v[c * CCH + t, :]
            w1 = p1_v[c * CCH + t, :]

            def body(i, _):
                for k in range(4):
                    off = i * 4 * NL + k * NL
                    Oc[t, pl.ds(off, NL)] = (Gc[2 * t, pl.ds(off, NL)] * w0
                                             + Gc[2 * t + 1, pl.ds(off, NL)] * w1)
                return 0

            lax.fori_loop(0, NVEC // 4, body, 0)
        sts[c] = pltpu.async_copy(
            Oc, out_hbm.at[pl.ds(base + c * CCH, CCH)], ssem[c % 2])
        if c + 2 < CNCH:
            cps[c + 2] = gather(c + 2)
    for c in range(max(0, CNCH - 2), CNCH):
        sts[c].wait()


@functools.cache
def _sc_combine_fn():
    return functools.partial(
        pl.kernel,
        mesh=plsc.VectorSubcoreMesh(core_axis_name="c", subcore_axis_name="s"),
        compiler_params=pltpu.CompilerParams(use_tc_tiling_on_sc=True),
        out_type=jax.ShapeDtypeStruct((S, D), jnp.float32),
        scratch_types=[
            pltpu.VMEM((2 * TPW,), jnp.int32),
            pltpu.VMEM((TPW, NL), jnp.float32),
            pltpu.VMEM((TPW, NL), jnp.float32),
            pltpu.VMEM((2 * CCH, D), jnp.float32),
            pltpu.VMEM((2 * CCH, D), jnp.float32),
            pltpu.VMEM((CCH, D), jnp.float32),
            pltpu.VMEM((CCH, D), jnp.float32),
            pltpu.SemaphoreType.DMA,
            pltpu.SemaphoreType.DMA,
            pltpu.SemaphoreType.DMA,
            pltpu.SemaphoreType.DMA,
        ],
    )(_sc_combine_body)


def kernel(x, Wg, bg, W1, b1, W2, b2):
    x2 = x.reshape(S, D)
    p0, p1, pp0, pp1, dlt, nt8, gs8 = _gate(x2, Wg, bg)
    ntiles, g_start = nt8[:, 0], gs8[:, 0]
    pad_base = jnp.arange(P, dtype=jnp.int32) & (S - 1)
    ppos_cat = jnp.concatenate([pp0, pp1])
    tok_pad = pad_base.at[ppos_cat].add(dlt, mode="promise_in_bounds",
                                        unique_indices=True)
    xs = _sc_gather_fn()(x2, tok_pad)
    ys = _ffn(ntiles, g_start, xs, W1, b1, W2, b2)
    p0b = jnp.broadcast_to(p0[:, None], (S, NL))
    p1b = jnp.broadcast_to(p1[:, None], (S, NL))
    pint = jnp.stack([pp0, pp1], axis=1).reshape(2 * S)
    out = _sc_combine_fn()(ys, pint, p0b, p1b)
    return out.reshape(x.shape)


# speedup vs baseline: 1.0092x; 1.0092x over previous
import functools

import jax
import jax.numpy as jnp
from jax import lax
from jax.experimental import pallas as pl
from jax.experimental.pallas import tpu as pltpu
from jax.experimental.pallas import tpu_sc as plsc

S, D, DFF, E, K = 2048, 1024, 4096, 8, 2
TR = 256
NT = (K * S) // TR + E
P = NT * TR
MAX_TILES_PER_E = S // TR
DFB = 1024
NJ = DFF // DFB

NC, NS, NL = 2, 16, 16
NW = NC * NS


def _gate_body(x_ref, wg_ref, bg_ref,
               p0_ref, p1_ref, pp0_ref, pp1_ref, dlt_ref, nt_ref, gs_ref):
    logits = lax.dot_general(
        wg_ref[...], x_ref[...], (((1,), (1,)), ((), ())),
        preferred_element_type=jnp.float32)
    logits = logits + bg_ref[:, 0:1]
    m0 = jnp.max(logits, axis=0)
    i0 = jnp.argmax(logits, axis=0)
    rows = lax.broadcasted_iota(jnp.int32, logits.shape, 0)
    masked = jnp.where(rows == i0[None, :], -1e30, logits)
    m1 = jnp.max(masked, axis=0)
    i1 = jnp.argmax(masked, axis=0)
    p0 = 1.0 / (1.0 + jnp.exp(m1 - m0))
    p0_ref[...] = p0
    p1_ref[...] = 1.0 - p0

    e_all = jnp.concatenate([i0, i1])
    oh = (e_all[None, :] == lax.broadcasted_iota(jnp.int32, (E, 2 * S), 0)
          ).astype(jnp.int32)
    c = oh
    sh = 1
    while sh < 2 * S:
        z = jnp.zeros((E, sh), jnp.int32)
        c = c + jnp.concatenate([z, c[:, : 2 * S - sh]], axis=1)
        sh *= 2
    rank = jnp.sum(oh * c, axis=0) - 1
    counts = c[:, -1:]
    ntiles = (counts + (TR - 1)) // TR
    tri = (lax.broadcasted_iota(jnp.int32, (E, E), 1)
           < lax.broadcasted_iota(jnp.int32, (E, E), 0)).astype(jnp.float32)
    ex = lax.dot_general(tri, ntiles.astype(jnp.float32),
                         (((1,), (0,)), ((), ())),
                         preferred_element_type=jnp.float32)
    g_start = (ex * TR).astype(jnp.int32)
    gsel = jnp.sum(oh * g_start, axis=0)
    ppos = gsel + rank
    tok = jnp.concatenate([jnp.arange(S, dtype=jnp.int32)] * 2)
    dlt_ref[...] = tok - (ppos & (S - 1))
    pp0_ref[...] = ppos[:S]
    pp1_ref[...] = ppos[S:]
    nt_ref[...] = jnp.broadcast_to(ntiles, (E, 128))
    gs_ref[...] = jnp.broadcast_to(g_start, (E, 128))


def _gate(x2, Wg, bg):
    bgb = jnp.broadcast_to(bg[:, None], (E, 128)).astype(jnp.float32)
    return pl.pallas_call(
        _gate_body,
        out_shape=(
            jax.ShapeDtypeStruct((S,), jnp.float32),
            jax.ShapeDtypeStruct((S,), jnp.float32),
            jax.ShapeDtypeStruct((S,), jnp.int32),
            jax.ShapeDtypeStruct((S,), jnp.int32),
            jax.ShapeDtypeStruct((2 * S,), jnp.int32),
            jax.ShapeDtypeStruct((E, 128), jnp.int32),
            jax.ShapeDtypeStruct((E, 128), jnp.int32),
        ),
    )(x2, Wg, bgb)


RPW = P // NW
GCH = 32
GNCH = RPW // GCH


def _sc_gather_body(x_hbm, idx_hbm, xs_hbm, idx_v, b0, b1,
                    g0, g1, st0, st1):
    wid = lax.axis_index("s") * NC + lax.axis_index("c")
    base = wid * RPW
    pltpu.sync_copy(idx_hbm.at[pl.ds(base, RPW)], idx_v)
    bufs, gsem, ssem = (b0, b1), (g0, g1), (st0, st1)

    def gather(c):
        return pltpu.async_copy(
            x_hbm.at[idx_v.at[pl.ds(c * GCH, GCH)]], bufs[c % 2], gsem[c % 2])

    cps = {0: gather(0), 1: gather(1)}
    sts = {}
    for c in range(GNCH):
        cps[c].wait()
        sts[c] = pltpu.async_copy(
            bufs[c % 2], xs_hbm.at[pl.ds(base + c * GCH, GCH)], ssem[c % 2])
        if 1 <= c < GNCH - 1:
            sts[c - 1].wait()
            cps[c + 1] = gather(c + 1)
    if GNCH >= 2:
        sts[GNCH - 2].wait()
    sts[GNCH - 1].wait()


@functools.cache
def _sc_gather_fn():
    return functools.partial(
        pl.kernel,
        mesh=plsc.VectorSubcoreMesh(core_axis_name="c", subcore_axis_name="s"),
        out_type=jax.ShapeDtypeStruct((P, D), jnp.float32),
        scratch_types=[
            pltpu.VMEM((RPW,), jnp.int32),
            pltpu.VMEM((GCH, D), jnp.float32),
            pltpu.VMEM((GCH, D), jnp.float32),
            pltpu.SemaphoreType.DMA,
            pltpu.SemaphoreType.DMA,
            pltpu.SemaphoreType.DMA,
            pltpu.SemaphoreType.DMA,
        ],
    )(_sc_gather_body)


def _ffn_body(ntiles_ref, gstart_ref, xs_ref, w1_ref, b1_ref, w2_ref, b2_ref,
              ys_ref, acc_ref, sem):
    e = pl.program_id(0)
    j = pl.program_id(1)
    nt = ntiles_ref[e]
    gs = pl.multiple_of(gstart_ref[e], TR)
    ep = jnp.maximum(e - 1, 0)
    ntp = ntiles_ref[ep]
    gsp = pl.multiple_of(gstart_ref[ep], TR)
    for t in range(MAX_TILES_PER_E):
        @pl.when((j == 0) & (e > 0) & (t < ntp))
        def _():
            pltpu.make_async_copy(
                acc_ref.at[t], ys_ref.at[pl.ds(gsp + t * TR, TR), :], sem
            ).wait()

        @pl.when(t < nt)
        def _():
            xt = xs_ref[pl.ds(gs + t * TR, TR), :]
            h = lax.dot_general(xt, w1_ref[...], (((1,), (1,)), ((), ())),
                                preferred_element_type=jnp.float32)
            h = h + b1_ref[0]
            h = 0.5 * h * (1.0 + lax.erf(h * 0.7071067811865476))
            y = lax.dot_general(h, w2_ref[...], (((1,), (1,)), ((), ())),
                                preferred_element_type=jnp.float32)

            @pl.when(j == 0)
            def _():
                acc_ref[t] = y + b2_ref[0]

            @pl.when(j > 0)
            def _():
                acc_ref[t] = acc_ref[t] + y

    @pl.when(j == NJ - 1)
    def _():
        for t in range(MAX_TILES_PER_E):
            @pl.when(t < nt)
            def _():
                pltpu.make_async_copy(
                    acc_ref.at[t], ys_ref.at[pl.ds(gs + t * TR, TR), :], sem
                ).start()
        @pl.when(e == E - 1)
        def _():
            for t in range(MAX_TILES_PER_E):
                @pl.when(t < nt)
                def _():
                    pltpu.make_async_copy(
                        acc_ref.at[t], ys_ref.at[pl.ds(gs + t * TR, TR), :], sem
                    ).wait()


def _ffn(ntiles, gstart, xs, W1, b1, W2, b2):
    b1r = b1.reshape(E, 1, DFF)
    b2r = b2.reshape(E, 1, D)
    grid_spec = pltpu.PrefetchScalarGridSpec(
        num_scalar_prefetch=2,
        grid=(E, NJ),
        in_specs=[
            pl.BlockSpec((P, D), lambda e, j, nt, gsr: (0, 0)),
            pl.BlockSpec((1, DFB, D), lambda e, j, nt, gsr: (e, j, 0)),
            pl.BlockSpec((1, 1, DFB), lambda e, j, nt, gsr: (e, 0, j)),
            pl.BlockSpec((1, D, DFB), lambda e, j, nt, gsr: (e, 0, j)),
            pl.BlockSpec((1, 1, D), lambda e, j, nt, gsr: (e, 0, 0)),
        ],
        out_specs=pl.BlockSpec(memory_space=pl.ANY),
        scratch_shapes=[
            pltpu.VMEM((MAX_TILES_PER_E, TR, D), jnp.float32),
            pltpu.SemaphoreType.DMA,
        ],
    )

    def body(ntiles_ref, gstart_ref, xs_ref, w1_ref, b1_ref, w2_ref, b2_ref,
             ys_ref, acc_ref, sem):
        _ffn_body(ntiles_ref, gstart_ref, xs_ref,
                  w1_ref.at[0], b1_ref.at[0], w2_ref.at[0], b2_ref.at[0],
                  ys_ref, acc_ref, sem)

    return pl.pallas_call(
        body,
        grid_spec=grid_spec,
        out_shape=jax.ShapeDtypeStruct((P, D), jnp.float32),
        compiler_params=pltpu.CompilerParams(
            dimension_semantics=("arbitrary", "arbitrary")),
    )(ntiles, gstart, xs, W1, b1r, W2, b2r)


TPW = S // NW
CCH = 16
CNCH = TPW // CCH
NVEC = D // NL


def _sc_combine_body(ys_hbm, pint_hbm, p0_hbm, p1_hbm, out_hbm,
                     ii_v, p0_v, p1_v, G0, G1, O0, O1, g0, g1, st0, st1):
    wid = lax.axis_index("s") * NC + lax.axis_index("c")
    base = wid * TPW
    pltpu.sync_copy(pint_hbm.at[pl.ds(2 * base, 2 * TPW)], ii_v)
    pltpu.sync_copy(p0_hbm.at[pl.ds(base, TPW), :], p0_v)
    pltpu.sync_copy(p1_hbm.at[pl.ds(base, TPW), :], p1_v)
    G, O, gsem, ssem = (G0, G1), (O0, O1), (g0, g1), (st0, st1)

    def gather(c):
        return pltpu.async_copy(
            ys_hbm.at[ii_v.at[pl.ds(c * 2 * CCH, 2 * CCH)]], G[c % 2],
            gsem[c % 2])

    cps = {0: gather(0), 1: gather(1)}
    sts = {}
    for c in range(CNCH):
        cps[c].wait()
        if c - 2 >= 0:
            sts[c - 2].wait()
        Gc, Oc = G[c % 2], O[c % 2]
        for t in range(CCH):
            w0 = p0_v[c * CCH + t, :]
            w1 = p1_v[c * CCH + t, :]

            def body(i, _):
                for k in range(4):
                    off = i * 4 * NL + k * NL
                    Oc[t, pl.ds(off, NL)] = (Gc[2 * t, pl.ds(off, NL)] * w0
                                             + Gc[2 * t + 1, pl.ds(off, NL)] * w1)
                return 0

            lax.fori_loop(0, NVEC // 4, body, 0)
        sts[c] = pltpu.async_copy(
            Oc, out_hbm.at[pl.ds(base + c * CCH, CCH)], ssem[c % 2])
        if c + 2 < CNCH:
            cps[c + 2] = gather(c + 2)
    for c in range(max(0, CNCH - 2), CNCH):
        sts[c].wait()


@functools.cache
def _sc_combine_fn():
    return functools.partial(
        pl.kernel,
        mesh=plsc.VectorSubcoreMesh(core_axis_name="c", subcore_axis_name="s"),
        out_type=jax.ShapeDtypeStruct((S, D), jnp.float32),
        scratch_types=[
            pltpu.VMEM((2 * TPW,), jnp.int32),
            pltpu.VMEM((TPW, NL), jnp.float32),
            pltpu.VMEM((TPW, NL), jnp.float32),
            pltpu.VMEM((2 * CCH, D), jnp.float32),
            pltpu.VMEM((2 * CCH, D), jnp.float32),
            pltpu.VMEM((CCH, D), jnp.float32),
            pltpu.VMEM((CCH, D), jnp.float32),
            pltpu.SemaphoreType.DMA,
            pltpu.SemaphoreType.DMA,
            pltpu.SemaphoreType.DMA,
            pltpu.SemaphoreType.DMA,
        ],
    )(_sc_combine_body)


def kernel(x, Wg, bg, W1, b1, W2, b2):
    x2 = x.reshape(S, D)
    p0, p1, pp0, pp1, dlt, nt8, gs8 = _gate(x2, Wg, bg)
    ntiles, g_start = nt8[:, 0], gs8[:, 0]
    pad_base = jnp.arange(P, dtype=jnp.int32) & (S - 1)
    ppos_cat = jnp.concatenate([pp0, pp1])
    tok_pad = pad_base.at[ppos_cat].add(dlt, mode="promise_in_bounds",
                                        unique_indices=True)
    xs = _sc_gather_fn()(x2, tok_pad)
    ys = _ffn(ntiles, g_start, xs, W1, b1, W2, b2)
    p0b = jnp.broadcast_to(p0[:, None], (S, NL))
    p1b = jnp.broadcast_to(p1[:, None], (S, NL))
    pint = jnp.stack([pp0, pp1], axis=1).reshape(2 * S)
    out = _sc_combine_fn()(ys, pint, p0b, p1b)
    return out.reshape(x.shape)
